# baseline (speedup 1.0000x reference)
import functools

import jax
import jax.numpy as jnp
from jax import lax
from jax.experimental import pallas as pl
from jax.experimental.pallas import tpu as pltpu
from jax.experimental.pallas import tpu_sc as plsc

D_IN = 64
D_OUT = 64
K = 64
B = 4096

NC = 2
NS = 16
NW = NC * NS
BPW = B // NW

PREP_BB = 512


def _slopes_body(y_ref, kn_ref, d_ref):
    y = y_ref[...]
    k0 = kn_ref[0]
    kN = kn_ref[K - 1]
    s = (kN - k0) / (K - 1)
    delta = (y[..., 1:] - y[..., :-1]) / (s + 1e-12)
    d0 = (3 * s * delta[..., 0] - s * delta[..., 1]) / (2 * s + 1e-12)
    dN = (3 * s * delta[..., -1] - s * delta[..., -2]) / (2 * s + 1e-12)

    def limit(di, deltai):
        di = jnp.where(di * deltai <= 0, jnp.zeros_like(di), di)
        di = jnp.where(jnp.abs(di) > 3 * jnp.abs(deltai), 3 * deltai, di)
        return di

    d0 = limit(d0, delta[..., 0])
    dN = limit(dN, delta[..., -1])
    dp = delta[..., :-1]
    dn = delta[..., 1:]
    same_sign = dp * dn > 0
    w = 3 * s
    d_int = (2 * w) / (w / (dp + 1e-12) + w / (dn + 1e-12) + 1e-12)
    d_int = jnp.where(same_sign, d_int, jnp.zeros_like(d_int))
    d_ref[...] = jnp.concatenate([d0[..., None], d_int, dN[..., None]], axis=-1)


def _prep_body(x_ref, kn_ref, ridx_ref, wts_ref):
    xb = x_ref[...]
    k0 = kn_ref[0]
    kN = kn_ref[K - 1]
    s = (kN - k0) / (K - 1)
    xc = jnp.clip(xb, k0, kN)
    u = (xc - k0) / s
    idxf = jnp.clip(jnp.floor(u), 0.0, K - 2)
    t = u - idxf
    t2 = t * t
    t3 = t2 * t
    h00 = 2 * t3 - 3 * t2 + 1
    h10 = t3 - 2 * t2 + t
    h01 = -2 * t3 + 3 * t2
    h11 = t3 - t2
    iio = lax.broadcasted_iota(jnp.int32, xb.shape, 1)
    ridx_ref[...] = iio * K + idxf.astype(jnp.int32)
    wts_ref[...] = jnp.concatenate(
        [h00[:, None, :], (h10 * s)[:, None, :],
         h01[:, None, :], (h11 * s)[:, None, :]], axis=1)


def _sc_body(ridx_hbm, wts_hbm, table_hbm, bias_hbm, out_hbm,
             idxv, wv, rows, outv, biasv, sem):
    cid = lax.axis_index("c")
    sid = lax.axis_index("s")
    wid = sid * NC + cid
    base = wid * BPW
    pltpu.sync_copy(ridx_hbm.at[pl.ds(base, BPW)], idxv)
    pltpu.sync_copy(wts_hbm.at[pl.ds(base, BPW)], wv)
    pltpu.sync_copy(bias_hbm, biasv)

    @pl.loop(0, BPW)
    def _b_loop(b):
        pltpu.async_copy(table_hbm.at[idxv.at[b]], rows, sem).wait()
        accs = [biasv[pl.ds(16 * c, 16)] for c in range(4)]
        for ci in range(D_IN // 16):
          w0v = wv[b, 0, pl.ds(ci * 16, 16)]
          w1v = wv[b, 1, pl.ds(ci * 16, 16)]
          w2v = wv[b, 2, pl.ds(ci * 16, 16)]
          w3v = wv[b, 3, pl.ds(ci * 16, 16)]
          for il in range(16):
            i = ci * 16 + il
            w0 = w0v[il]
            w1 = w1v[il]
            w2 = w2v[il]
            w3 = w3v[il]
            for c in range(4):
              r0 = rows[i, pl.ds(c * 16, 16)]
              r1 = rows[i, pl.ds(64 + c * 16, 16)]
              r2 = rows[i, pl.ds(128 + c * 16, 16)]
              r3 = rows[i, pl.ds(192 + c * 16, 16)]
              accs[c] = accs[c] + w0 * r0 + w1 * r1 + w2 * r2 + w3 * r3
        for c in range(4):
            outv[b, pl.ds(16 * c, 16)] = accs[c]

    pltpu.sync_copy(outv, out_hbm.at[pl.ds(base, BPW)])


def kernel(x, coeffs, bias, knots):
    slopes = pl.pallas_call(
        _slopes_body,
        out_shape=jax.ShapeDtypeStruct((D_OUT, D_IN, K), jnp.float32),
        in_specs=[
            pl.BlockSpec(memory_space=pltpu.VMEM),
            pl.BlockSpec(memory_space=pltpu.SMEM),
        ],
        out_specs=pl.BlockSpec(memory_space=pltpu.VMEM),
    )(coeffs, knots)

    ridx, wts = pl.pallas_call(
        _prep_body,
        grid=(B // PREP_BB,),
        in_specs=[
            pl.BlockSpec((PREP_BB, D_IN), lambda j: (j, 0)),
            pl.BlockSpec(memory_space=pltpu.SMEM),
        ],
        out_specs=[
            pl.BlockSpec((PREP_BB, D_IN), lambda j: (j, 0)),
            pl.BlockSpec((PREP_BB, 4, D_IN), lambda j: (j, 0, 0)),
        ],
        out_shape=[
            jax.ShapeDtypeStruct((B, D_IN), jnp.int32),
            jax.ShapeDtypeStruct((B, 4, D_IN), jnp.float32),
        ],
    )(x, knots)

    yio = coeffs.transpose(1, 2, 0)
    dio = slopes.transpose(1, 2, 0)
    yn = jnp.roll(yio, -1, axis=1)
    dn = jnp.roll(dio, -1, axis=1)
    table = jnp.concatenate([yio, dio, yn, dn], axis=2).reshape(D_IN * K, 4 * D_OUT)

    sc_call = pl.kernel(
        _sc_body,
        out_type=jax.ShapeDtypeStruct((B, D_OUT), jnp.float32),
        mesh=plsc.VectorSubcoreMesh(
            core_axis_name="c", subcore_axis_name="s",
            num_cores=NC, num_subcores=NS),
        scratch_types=[
            pltpu.VMEM((BPW, D_IN), jnp.int32),
            pltpu.VMEM((BPW, 4, D_IN), jnp.float32),
            pltpu.VMEM((D_IN, 4 * D_OUT), jnp.float32),
            pltpu.VMEM((BPW, D_OUT), jnp.float32),
            pltpu.VMEM((D_OUT,), jnp.float32),
            pltpu.SemaphoreType.DMA,
        ],
    )
    return sc_call(ridx, wts, table, bias)


# speedup vs baseline: 1.2338x; 1.2338x over previous
import functools

import jax
import jax.numpy as jnp
from jax import lax
from jax.experimental import pallas as pl
from jax.experimental.pallas import tpu as pltpu
from jax.experimental.pallas import tpu_sc as plsc

D_IN = 64
D_OUT = 64
K = 64
B = 4096

NC = 2
NS = 16
NW = NC * NS
BPW = B // NW

PREP_BB = 512


def _slopes_body(y_ref, kn_ref, d_ref):
    y = y_ref[...]
    k0 = kn_ref[0]
    kN = kn_ref[K - 1]
    s = (kN - k0) / (K - 1)
    delta = (y[..., 1:] - y[..., :-1]) / (s + 1e-12)
    d0 = (3 * s * delta[..., 0] - s * delta[..., 1]) / (2 * s + 1e-12)
    dN = (3 * s * delta[..., -1] - s * delta[..., -2]) / (2 * s + 1e-12)

    def limit(di, deltai):
        di = jnp.where(di * deltai <= 0, jnp.zeros_like(di), di)
        di = jnp.where(jnp.abs(di) > 3 * jnp.abs(deltai), 3 * deltai, di)
        return di

    d0 = limit(d0, delta[..., 0])
    dN = limit(dN, delta[..., -1])
    dp = delta[..., :-1]
    dn = delta[..., 1:]
    same_sign = dp * dn > 0
    w = 3 * s
    d_int = (2 * w) / (w / (dp + 1e-12) + w / (dn + 1e-12) + 1e-12)
    d_int = jnp.where(same_sign, d_int, jnp.zeros_like(d_int))
    d_ref[...] = jnp.concatenate([d0[..., None], d_int, dN[..., None]], axis=-1)


def _prep_body(x_ref, kn_ref, ridx_ref, wts_ref):
    xb = x_ref[...]
    k0 = kn_ref[0]
    kN = kn_ref[K - 1]
    s = (kN - k0) / (K - 1)
    xc = jnp.clip(xb, k0, kN)
    u = (xc - k0) / s
    idxf = jnp.clip(jnp.floor(u), 0.0, K - 2)
    t = u - idxf
    t2 = t * t
    t3 = t2 * t
    h00 = 2 * t3 - 3 * t2 + 1
    h10 = t3 - 2 * t2 + t
    h11 = t3 - t2
    iio = lax.broadcasted_iota(jnp.int32, xb.shape, 1)
    ridx_ref[...] = iio * K + idxf.astype(jnp.int32)
    wts_ref[...] = jnp.concatenate(
        [h00[:, None, :], (h10 * s)[:, None, :], (h11 * s)[:, None, :]], axis=1)


HALF = D_IN // 2


def _accum_half(b, half, rows, wv, biasv, outr):
    if half == 0:
        accs = [biasv[pl.ds(16 * c, 16)] for c in range(4)]
    else:
        accs = [outr[pl.ds(16 * c, 16)] for c in range(4)]
    for ci in range(HALF // 16):
        w0v = wv[b, 0, pl.ds(half * HALF + ci * 16, 16)]
        w1v = wv[b, 1, pl.ds(half * HALF + ci * 16, 16)]
        w2v = wv[b, 2, pl.ds(half * HALF + ci * 16, 16)]
        for il in range(16):
            i = ci * 16 + il
            w0 = w0v[il]
            w1 = w1v[il]
            w2 = w2v[il]
            for c in range(4):
                r0 = rows[i, pl.ds(c * 16, 16)]
                r1 = rows[i, pl.ds(64 + c * 16, 16)]
                r2 = rows[i, pl.ds(128 + c * 16, 16)]
                r3 = rows[i, pl.ds(192 + c * 16, 16)]
                accs[c] = accs[c] + r0 + w0 * r1 + w1 * r2 + w2 * r3
    for c in range(4):
        outr[pl.ds(16 * c, 16)] = accs[c]


def _sc_body(ridx2_hbm, wts_hbm, table_hbm, bias_hbm, out_hbm,
             idxv, wv, rows0, rows1, outr, biasv, sem0, sem1, semo):
    cid = lax.axis_index("c")
    sid = lax.axis_index("s")
    wid = sid * NC + cid
    base = wid * BPW
    pltpu.sync_copy(ridx2_hbm.at[pl.ds(2 * base, 2 * BPW)], idxv)
    pltpu.sync_copy(wts_hbm.at[pl.ds(base, BPW)], wv)
    pltpu.sync_copy(bias_hbm, biasv)

    pltpu.async_copy(table_hbm.at[idxv.at[0]], rows0, sem0)

    @pl.loop(0, 2 * BPW, step=2)
    def _g_loop(g):
        b = g // 2
        pltpu.async_copy(table_hbm.at[idxv.at[g + 1]], rows1, sem1)
        pltpu.make_async_copy(table_hbm.at[idxv.at[g]], rows0, sem0).wait()

        @pl.when(g > 0)
        def _drain_prev():
            pltpu.make_async_copy(outr, out_hbm.at[base + b - 1], semo).wait()

        _accum_half(b, 0, rows0, wv, biasv, outr)

        @pl.when(g + 2 < 2 * BPW)
        def _issue_next():
            pltpu.async_copy(table_hbm.at[idxv.at[g + 2]], rows0, sem0)

        pltpu.make_async_copy(table_hbm.at[idxv.at[g + 1]], rows1, sem1).wait()
        _accum_half(b, 1, rows1, wv, biasv, outr)
        pltpu.async_copy(outr, out_hbm.at[base + b], semo)

    pltpu.make_async_copy(outr, out_hbm.at[base + BPW - 1], semo).wait()


def kernel(x, coeffs, bias, knots):
    slopes = pl.pallas_call(
        _slopes_body,
        out_shape=jax.ShapeDtypeStruct((D_OUT, D_IN, K), jnp.float32),
        in_specs=[
            pl.BlockSpec(memory_space=pltpu.VMEM),
            pl.BlockSpec(memory_space=pltpu.SMEM),
        ],
        out_specs=pl.BlockSpec(memory_space=pltpu.VMEM),
    )(coeffs, knots)

    ridx, wts = pl.pallas_call(
        _prep_body,
        grid=(B // PREP_BB,),
        in_specs=[
            pl.BlockSpec((PREP_BB, D_IN), lambda j: (j, 0)),
            pl.BlockSpec(memory_space=pltpu.SMEM),
        ],
        out_specs=[
            pl.BlockSpec((PREP_BB, D_IN), lambda j: (j, 0)),
            pl.BlockSpec((PREP_BB, 3, D_IN), lambda j: (j, 0, 0)),
        ],
        out_shape=[
            jax.ShapeDtypeStruct((B, D_IN), jnp.int32),
            jax.ShapeDtypeStruct((B, 3, D_IN), jnp.float32),
        ],
    )(x, knots)

    yio = coeffs.transpose(1, 2, 0)
    dio = slopes.transpose(1, 2, 0)
    yn = jnp.roll(yio, -1, axis=1)
    dn = jnp.roll(dio, -1, axis=1)
    table = jnp.concatenate([yn, yio - yn, dio, dn], axis=2).reshape(
        D_IN * K, 4 * D_OUT)

    sc_call = pl.kernel(
        _sc_body,
        out_type=jax.ShapeDtypeStruct((B, D_OUT), jnp.float32),
        mesh=plsc.VectorSubcoreMesh(
            core_axis_name="c", subcore_axis_name="s",
            num_cores=NC, num_subcores=NS),
        scratch_types=[
            pltpu.VMEM((2 * BPW, HALF), jnp.int32),
            pltpu.VMEM((BPW, 3, D_IN), jnp.float32),
            pltpu.VMEM((HALF, 4 * D_OUT), jnp.float32),
            pltpu.VMEM((HALF, 4 * D_OUT), jnp.float32),
            pltpu.VMEM((D_OUT,), jnp.float32),
            pltpu.VMEM((D_OUT,), jnp.float32),
            pltpu.SemaphoreType.DMA,
            pltpu.SemaphoreType.DMA,
            pltpu.SemaphoreType.DMA,
        ],
    )
    return sc_call(ridx.reshape(2 * B, HALF), wts, table, bias)


# speedup vs baseline: 2.3676x; 1.9190x over previous
import functools

import jax
import jax.numpy as jnp
from jax import lax
from jax.experimental import pallas as pl
from jax.experimental.pallas import tpu as pltpu
from jax.experimental.pallas import tpu_sc as plsc

D_IN = 64
D_OUT = 64
K = 64
B = 4096

NC = 2
NS = 16
NW = NC * NS
BPW = B // NW

PREP_BB = 512


def _slopes_body(y_ref, kn_ref, d_ref):
    y = y_ref[...]
    k0 = kn_ref[0]
    kN = kn_ref[K - 1]
    s = (kN - k0) / (K - 1)
    delta = (y[..., 1:] - y[..., :-1]) / (s + 1e-12)
    d0 = (3 * s * delta[..., 0] - s * delta[..., 1]) / (2 * s + 1e-12)
    dN = (3 * s * delta[..., -1] - s * delta[..., -2]) / (2 * s + 1e-12)

    def limit(di, deltai):
        di = jnp.where(di * deltai <= 0, jnp.zeros_like(di), di)
        di = jnp.where(jnp.abs(di) > 3 * jnp.abs(deltai), 3 * deltai, di)
        return di

    d0 = limit(d0, delta[..., 0])
    dN = limit(dN, delta[..., -1])
    dp = delta[..., :-1]
    dn = delta[..., 1:]
    same_sign = dp * dn > 0
    w = 3 * s
    d_int = (2 * w) / (w / (dp + 1e-12) + w / (dn + 1e-12) + 1e-12)
    d_int = jnp.where(same_sign, d_int, jnp.zeros_like(d_int))
    d_ref[...] = jnp.concatenate([d0[..., None], d_int, dN[..., None]], axis=-1)


def _prep_body(x_ref, kn_ref, ridx_ref, wts_ref):
    xb = x_ref[...]
    k0 = kn_ref[0]
    kN = kn_ref[K - 1]
    s = (kN - k0) / (K - 1)
    xc = jnp.clip(xb, k0, kN)
    u = (xc - k0) / s
    idxf = jnp.clip(jnp.floor(u), 0.0, K - 2)
    t = u - idxf
    t2 = t * t
    t3 = t2 * t
    h00 = 2 * t3 - 3 * t2 + 1
    h10 = t3 - 2 * t2 + t
    h11 = t3 - t2
    iio = lax.broadcasted_iota(jnp.int32, xb.shape, 1)
    ridx_ref[...] = iio * K + idxf.astype(jnp.int32)
    wts_ref[...] = jnp.concatenate(
        [h00[:, None, :], (h10 * s)[:, None, :], (h11 * s)[:, None, :]], axis=1)


HALF = D_IN // 2


def _accum_half(b, half, rows, wv, biasv, outr):
    zero = jnp.zeros((16,), jnp.float32)
    if half == 0:
        init = [[biasv[pl.ds(16 * c, 16)], zero] for c in range(4)]
    else:
        init = [[outr[pl.ds(16 * c, 16)], zero] for c in range(4)]

    @pl.loop(0, HALF // 16, init_carry=init)
    def _ci_loop(ci, accs):
        w0v = wv[b, 0, pl.ds(half * HALF + ci * 16, 16)]
        w1v = wv[b, 1, pl.ds(half * HALF + ci * 16, 16)]
        w2v = wv[b, 2, pl.ds(half * HALF + ci * 16, 16)]
        ibase = ci * 16
        for il in range(16):
            p = il % 2
            w0 = w0v[il]
            w1 = w1v[il]
            w2 = w2v[il]
            for c in range(4):
                r0 = rows[ibase + il, pl.ds(c * 16, 16)]
                r1 = rows[ibase + il, pl.ds(64 + c * 16, 16)]
                r2 = rows[ibase + il, pl.ds(128 + c * 16, 16)]
                r3 = rows[ibase + il, pl.ds(192 + c * 16, 16)]
                s1 = r0 + w0 * r1
                s2 = w1 * r2 + w2 * r3
                accs[c][p] = accs[c][p] + (s1 + s2)
        return accs

    accs = _ci_loop
    for c in range(4):
        outr[pl.ds(16 * c, 16)] = accs[c][0] + accs[c][1]


def _sc_body(ridx2_hbm, wts_hbm, table_hbm, bias_hbm, out_hbm,
             idxv, wv, rows0, rows1, outr, biasv, sem0, sem1, semo):
    cid = lax.axis_index("c")
    sid = lax.axis_index("s")
    wid = sid * NC + cid
    base = wid * BPW
    pltpu.sync_copy(ridx2_hbm.at[pl.ds(2 * base, 2 * BPW)], idxv)
    pltpu.sync_copy(wts_hbm.at[pl.ds(base, BPW)], wv)
    pltpu.sync_copy(bias_hbm, biasv)

    pltpu.async_copy(table_hbm.at[idxv.at[0]], rows0, sem0)

    @pl.loop(0, 2 * BPW, step=2)
    def _g_loop(g):
        b = g // 2
        pltpu.async_copy(table_hbm.at[idxv.at[g + 1]], rows1, sem1)
        pltpu.make_async_copy(table_hbm.at[idxv.at[g]], rows0, sem0).wait()

        @pl.when(g > 0)
        def _drain_prev():
            pltpu.make_async_copy(outr, out_hbm.at[base + b - 1], semo).wait()

        _accum_half(b, 0, rows0, wv, biasv, outr)

        @pl.when(g + 2 < 2 * BPW)
        def _issue_next():
            pltpu.async_copy(table_hbm.at[idxv.at[g + 2]], rows0, sem0)

        pltpu.make_async_copy(table_hbm.at[idxv.at[g + 1]], rows1, sem1).wait()
        _accum_half(b, 1, rows1, wv, biasv, outr)
        pltpu.async_copy(outr, out_hbm.at[base + b], semo)

    pltpu.make_async_copy(outr, out_hbm.at[base + BPW - 1], semo).wait()


def kernel(x, coeffs, bias, knots):
    slopes = pl.pallas_call(
        _slopes_body,
        out_shape=jax.ShapeDtypeStruct((D_OUT, D_IN, K), jnp.float32),
        in_specs=[
            pl.BlockSpec(memory_space=pltpu.VMEM),
            pl.BlockSpec(memory_space=pltpu.SMEM),
        ],
        out_specs=pl.BlockSpec(memory_space=pltpu.VMEM),
    )(coeffs, knots)

    ridx, wts = pl.pallas_call(
        _prep_body,
        grid=(B // PREP_BB,),
        in_specs=[
            pl.BlockSpec((PREP_BB, D_IN), lambda j: (j, 0)),
            pl.BlockSpec(memory_space=pltpu.SMEM),
        ],
        out_specs=[
            pl.BlockSpec((PREP_BB, D_IN), lambda j: (j, 0)),
            pl.BlockSpec((PREP_BB, 3, D_IN), lambda j: (j, 0, 0)),
        ],
        out_shape=[
            jax.ShapeDtypeStruct((B, D_IN), jnp.int32),
            jax.ShapeDtypeStruct((B, 3, D_IN), jnp.float32),
        ],
    )(x, knots)

    yio = coeffs.transpose(1, 2, 0)
    dio = slopes.transpose(1, 2, 0)
    yn = jnp.roll(yio, -1, axis=1)
    dn = jnp.roll(dio, -1, axis=1)
    table = jnp.concatenate([yn, yio - yn, dio, dn], axis=2).reshape(
        D_IN * K, 4 * D_OUT)

    sc_call = pl.kernel(
        _sc_body,
        out_type=jax.ShapeDtypeStruct((B, D_OUT), jnp.float32),
        mesh=plsc.VectorSubcoreMesh(
            core_axis_name="c", subcore_axis_name="s",
            num_cores=NC, num_subcores=NS),
        scratch_types=[
            pltpu.VMEM((2 * BPW, HALF), jnp.int32),
            pltpu.VMEM((BPW, 3, D_IN), jnp.float32),
            pltpu.VMEM((HALF, 4 * D_OUT), jnp.float32),
            pltpu.VMEM((HALF, 4 * D_OUT), jnp.float32),
            pltpu.VMEM((D_OUT,), jnp.float32),
            pltpu.VMEM((D_OUT,), jnp.float32),
            pltpu.SemaphoreType.DMA,
            pltpu.SemaphoreType.DMA,
            pltpu.SemaphoreType.DMA,
        ],
    )
    return sc_call(ridx.reshape(2 * B, HALF), wts, table, bias)


# speedup vs baseline: 5.6935x; 2.4047x over previous
import functools

import jax
import jax.numpy as jnp
from jax import lax
from jax.experimental import pallas as pl
from jax.experimental.pallas import tpu as pltpu
from jax.experimental.pallas import tpu_sc as plsc

D_IN = 64
D_OUT = 64
K = 64
B = 4096

NC = 2
NS = 16
NW = NC * NS

BSC = 1024
BTC = B - BSC
BPW = BSC // NW

PREP_BB = 512
TC_BB = 512


def _slopes_body(y_ref, kn_ref, d_ref):
    y = y_ref[...]
    k0 = kn_ref[0]
    kN = kn_ref[K - 1]
    s = (kN - k0) / (K - 1)
    delta = (y[..., 1:] - y[..., :-1]) / (s + 1e-12)
    d0 = (3 * s * delta[..., 0] - s * delta[..., 1]) / (2 * s + 1e-12)
    dN = (3 * s * delta[..., -1] - s * delta[..., -2]) / (2 * s + 1e-12)

    def limit(di, deltai):
        di = jnp.where(di * deltai <= 0, jnp.zeros_like(di), di)
        di = jnp.where(jnp.abs(di) > 3 * jnp.abs(deltai), 3 * deltai, di)
        return di

    d0 = limit(d0, delta[..., 0])
    dN = limit(dN, delta[..., -1])
    dp = delta[..., :-1]
    dn = delta[..., 1:]
    same_sign = dp * dn > 0
    w = 3 * s
    d_int = (2 * w) / (w / (dp + 1e-12) + w / (dn + 1e-12) + 1e-12)
    d_int = jnp.where(same_sign, d_int, jnp.zeros_like(d_int))
    d_ref[...] = jnp.concatenate([d0[..., None], d_int, dN[..., None]], axis=-1)


def _prep_body(x_ref, kn_ref, ridx_ref, wts_ref):
    xb = x_ref[...]
    k0 = kn_ref[0]
    kN = kn_ref[K - 1]
    s = (kN - k0) / (K - 1)
    xc = jnp.clip(xb, k0, kN)
    u = (xc - k0) / s
    idxf = jnp.clip(jnp.floor(u), 0.0, K - 2)
    t = u - idxf
    t2 = t * t
    t3 = t2 * t
    h00 = 2 * t3 - 3 * t2 + 1
    h10 = t3 - 2 * t2 + t
    h11 = t3 - t2
    iio = lax.broadcasted_iota(jnp.int32, xb.shape, 1)
    ridx_ref[...] = iio * K + idxf.astype(jnp.int32)
    wts_ref[...] = jnp.concatenate(
        [h00[:, None, :], (h10 * s)[:, None, :], (h11 * s)[:, None, :]], axis=1)


HALF = D_IN // 2


def _accum_half(b, half, rows, wv, biasv, outr):
    zero = jnp.zeros((16,), jnp.float32)
    if half == 0:
        init = [[biasv[pl.ds(16 * c, 16)], zero] for c in range(4)]
    else:
        init = [[outr[pl.ds(16 * c, 16)], zero] for c in range(4)]

    @pl.loop(0, HALF // 16, init_carry=init)
    def _ci_loop(ci, accs):
        w0v = wv[b, 0, pl.ds(half * HALF + ci * 16, 16)]
        w1v = wv[b, 1, pl.ds(half * HALF + ci * 16, 16)]
        w2v = wv[b, 2, pl.ds(half * HALF + ci * 16, 16)]
        ibase = ci * 16
        for il in range(16):
            p = il % 2
            w0 = w0v[il]
            w1 = w1v[il]
            w2 = w2v[il]
            for c in range(4):
                r0 = rows[ibase + il, pl.ds(c * 16, 16)]
                r1 = rows[ibase + il, pl.ds(64 + c * 16, 16)]
                r2 = rows[ibase + il, pl.ds(128 + c * 16, 16)]
                r3 = rows[ibase + il, pl.ds(192 + c * 16, 16)]
                s1 = r0 + w0 * r1
                s2 = w1 * r2 + w2 * r3
                accs[c][p] = accs[c][p] + (s1 + s2)
        return accs

    accs = _ci_loop
    for c in range(4):
        outr[pl.ds(16 * c, 16)] = accs[c][0] + accs[c][1]


def _sc_body(ridx2_hbm, wts_hbm, table_hbm, bias_hbm, out_hbm,
             idxv, wv, rows0, rows1, outr, biasv, sem0, sem1, semo):
    cid = lax.axis_index("c")
    sid = lax.axis_index("s")
    wid = sid * NC + cid
    base = wid * BPW
    pltpu.sync_copy(ridx2_hbm.at[pl.ds(2 * base, 2 * BPW)], idxv)
    pltpu.sync_copy(wts_hbm.at[pl.ds(base, BPW)], wv)
    pltpu.sync_copy(bias_hbm, biasv)

    pltpu.async_copy(table_hbm.at[idxv.at[0]], rows0, sem0)

    @pl.loop(0, 2 * BPW, step=2)
    def _g_loop(g):
        b = g // 2
        pltpu.async_copy(table_hbm.at[idxv.at[g + 1]], rows1, sem1)
        pltpu.make_async_copy(table_hbm.at[idxv.at[g]], rows0, sem0).wait()

        @pl.when(g > 0)
        def _drain_prev():
            pltpu.make_async_copy(outr, out_hbm.at[base + b - 1], semo).wait()

        _accum_half(b, 0, rows0, wv, biasv, outr)

        @pl.when(g + 2 < 2 * BPW)
        def _issue_next():
            pltpu.async_copy(table_hbm.at[idxv.at[g + 2]], rows0, sem0)

        pltpu.make_async_copy(table_hbm.at[idxv.at[g + 1]], rows1, sem1).wait()
        _accum_half(b, 1, rows1, wv, biasv, outr)
        pltpu.async_copy(outr, out_hbm.at[base + b], semo)

    pltpu.make_async_copy(outr, out_hbm.at[base + BPW - 1], semo).wait()


def _tc_eval_body(xT_ref, y2_ref, d2_ref, kn_ref, b_ref, out_ref):
    xb = xT_ref[...]
    k0 = kn_ref[0]
    kN = kn_ref[K - 1]
    s = (kN - k0) / (K - 1)
    xc = jnp.clip(xb, k0, kN)
    u = (xc - k0) / s
    idxf = jnp.clip(jnp.floor(u), 0.0, K - 2)
    t = u - idxf
    t2 = t * t
    t3 = t2 * t
    h00 = 2 * t3 - 3 * t2 + 1
    h10 = t3 - 2 * t2 + t
    h01 = -2 * t3 + 3 * t2
    h11 = t3 - t2
    a0 = h00
    a1 = h01
    b0 = h10 * s
    b1 = h11 * s
    kio = jax.lax.broadcasted_iota(jnp.int32, (D_IN, K, TC_BB), 1)
    idxe = idxf.astype(jnp.int32)[:, None, :]
    e0 = kio == idxe
    e1 = kio == idxe + 1
    zero = jnp.zeros((), jnp.float32)
    W0 = jnp.where(e0, a0[:, None, :], zero) + jnp.where(e1, a1[:, None, :], zero)
    W1 = jnp.where(e0, b0[:, None, :], zero) + jnp.where(e1, b1[:, None, :], zero)
    W0 = W0.reshape(D_IN * K, TC_BB)
    W1 = W1.reshape(D_IN * K, TC_BB)
    acc = jnp.dot(y2_ref[...], W0, preferred_element_type=jnp.float32)
    acc = acc + jnp.dot(d2_ref[...], W1, preferred_element_type=jnp.float32)
    out_ref[...] = acc + b_ref[...]


def kernel(x, coeffs, bias, knots):
    slopes = pl.pallas_call(
        _slopes_body,
        out_shape=jax.ShapeDtypeStruct((D_OUT, D_IN, K), jnp.float32),
        in_specs=[
            pl.BlockSpec(memory_space=pltpu.VMEM),
            pl.BlockSpec(memory_space=pltpu.SMEM),
        ],
        out_specs=pl.BlockSpec(memory_space=pltpu.VMEM),
    )(coeffs, knots)

    ridx, wts = pl.pallas_call(
        _prep_body,
        grid=(BSC // PREP_BB,),
        in_specs=[
            pl.BlockSpec((PREP_BB, D_IN), lambda j: (j, 0)),
            pl.BlockSpec(memory_space=pltpu.SMEM),
        ],
        out_specs=[
            pl.BlockSpec((PREP_BB, D_IN), lambda j: (j, 0)),
            pl.BlockSpec((PREP_BB, 3, D_IN), lambda j: (j, 0, 0)),
        ],
        out_shape=[
            jax.ShapeDtypeStruct((BSC, D_IN), jnp.int32),
            jax.ShapeDtypeStruct((BSC, 3, D_IN), jnp.float32),
        ],
    )(x[:BSC], knots)

    yio = coeffs.transpose(1, 2, 0)
    dio = slopes.transpose(1, 2, 0)
    yn = jnp.roll(yio, -1, axis=1)
    dn = jnp.roll(dio, -1, axis=1)
    table = jnp.concatenate([yn, yio - yn, dio, dn], axis=2).reshape(
        D_IN * K, 4 * D_OUT)

    sc_call = pl.kernel(
        _sc_body,
        out_type=jax.ShapeDtypeStruct((BSC, D_OUT), jnp.float32),
        mesh=plsc.VectorSubcoreMesh(
            core_axis_name="c", subcore_axis_name="s",
            num_cores=NC, num_subcores=NS),
        scratch_types=[
            pltpu.VMEM((2 * BPW, HALF), jnp.int32),
            pltpu.VMEM((BPW, 3, D_IN), jnp.float32),
            pltpu.VMEM((HALF, 4 * D_OUT), jnp.float32),
            pltpu.VMEM((HALF, 4 * D_OUT), jnp.float32),
            pltpu.VMEM((D_OUT,), jnp.float32),
            pltpu.VMEM((D_OUT,), jnp.float32),
            pltpu.SemaphoreType.DMA,
            pltpu.SemaphoreType.DMA,
            pltpu.SemaphoreType.DMA,
        ],
    )
    sc_out = sc_call(ridx.reshape(2 * BSC, HALF), wts, table, bias)

    xT = x[BSC:].T
    y2 = coeffs.reshape(D_OUT, D_IN * K)
    d2 = slopes.reshape(D_OUT, D_IN * K)
    bias2 = bias.reshape(D_OUT, 1)
    tc_outT = pl.pallas_call(
        _tc_eval_body,
        grid=(BTC // TC_BB,),
        in_specs=[
            pl.BlockSpec((D_IN, TC_BB), lambda j: (0, j)),
            pl.BlockSpec((D_OUT, D_IN * K), lambda j: (0, 0)),
            pl.BlockSpec((D_OUT, D_IN * K), lambda j: (0, 0)),
            pl.BlockSpec(memory_space=pltpu.SMEM),
            pl.BlockSpec((D_OUT, 1), lambda j: (0, 0)),
        ],
        out_specs=pl.BlockSpec((D_OUT, TC_BB), lambda j: (0, j)),
        out_shape=jax.ShapeDtypeStruct((D_OUT, BTC), jnp.float32),
    )(xT, y2, d2, knots, bias2)
    return jnp.concatenate([sc_out, tc_outT.T], axis=0)


# speedup vs baseline: 7.1973x; 1.2641x over previous
import functools

import jax
import jax.numpy as jnp
from jax import lax
from jax.experimental import pallas as pl
from jax.experimental.pallas import tpu as pltpu
from jax.experimental.pallas import tpu_sc as plsc

D_IN = 64
D_OUT = 64
K = 64
B = 4096

NC = 2
NS = 16
NW = NC * NS

BSC = 512
BTC = B - BSC
BPW = BSC // NW

PREP_BB = 512
TC_BB = 512


def _slopes_body(y_ref, kn_ref, d_ref):
    y = y_ref[...]
    k0 = kn_ref[0]
    kN = kn_ref[K - 1]
    s = (kN - k0) / (K - 1)
    delta = (y[..., 1:] - y[..., :-1]) / (s + 1e-12)
    d0 = (3 * s * delta[..., 0] - s * delta[..., 1]) / (2 * s + 1e-12)
    dN = (3 * s * delta[..., -1] - s * delta[..., -2]) / (2 * s + 1e-12)

    def limit(di, deltai):
        di = jnp.where(di * deltai <= 0, jnp.zeros_like(di), di)
        di = jnp.where(jnp.abs(di) > 3 * jnp.abs(deltai), 3 * deltai, di)
        return di

    d0 = limit(d0, delta[..., 0])
    dN = limit(dN, delta[..., -1])
    dp = delta[..., :-1]
    dn = delta[..., 1:]
    same_sign = dp * dn > 0
    w = 3 * s
    d_int = (2 * w) / (w / (dp + 1e-12) + w / (dn + 1e-12) + 1e-12)
    d_int = jnp.where(same_sign, d_int, jnp.zeros_like(d_int))
    d_ref[...] = jnp.concatenate([d0[..., None], d_int, dN[..., None]], axis=-1)


def _prep_body(x_ref, kn_ref, ridx_ref, wts_ref):
    xb = x_ref[...]
    k0 = kn_ref[0]
    kN = kn_ref[K - 1]
    s = (kN - k0) / (K - 1)
    xc = jnp.clip(xb, k0, kN)
    u = (xc - k0) / s
    idxf = jnp.clip(jnp.floor(u), 0.0, K - 2)
    t = u - idxf
    t2 = t * t
    t3 = t2 * t
    h00 = 2 * t3 - 3 * t2 + 1
    h10 = t3 - 2 * t2 + t
    h11 = t3 - t2
    iio = lax.broadcasted_iota(jnp.int32, xb.shape, 1)
    ridx_ref[...] = iio * K + idxf.astype(jnp.int32)
    wts_ref[...] = jnp.concatenate(
        [h00[:, None, :], (h10 * s)[:, None, :], (h11 * s)[:, None, :]], axis=1)


HALF = D_IN // 2


def _accum_half(b, half, rows, wv, biasv, outr):
    zero = jnp.zeros((16,), jnp.float32)
    if half == 0:
        init = [[biasv[pl.ds(16 * c, 16)], zero] for c in range(4)]
    else:
        init = [[outr[pl.ds(16 * c, 16)], zero] for c in range(4)]

    @pl.loop(0, HALF // 16, init_carry=init)
    def _ci_loop(ci, accs):
        w0v = wv[b, 0, pl.ds(half * HALF + ci * 16, 16)]
        w1v = wv[b, 1, pl.ds(half * HALF + ci * 16, 16)]
        w2v = wv[b, 2, pl.ds(half * HALF + ci * 16, 16)]
        ibase = ci * 16
        for il in range(16):
            p = il % 2
            w0 = w0v[il]
            w1 = w1v[il]
            w2 = w2v[il]
            for c in range(4):
                r0 = rows[ibase + il, pl.ds(c * 16, 16)]
                r1 = rows[ibase + il, pl.ds(64 + c * 16, 16)]
                r2 = rows[ibase + il, pl.ds(128 + c * 16, 16)]
                r3 = rows[ibase + il, pl.ds(192 + c * 16, 16)]
                s1 = r0 + w0 * r1
                s2 = w1 * r2 + w2 * r3
                accs[c][p] = accs[c][p] + (s1 + s2)
        return accs

    accs = _ci_loop
    for c in range(4):
        outr[pl.ds(16 * c, 16)] = accs[c][0] + accs[c][1]


def _sc_body(ridx2_hbm, wts_hbm, table_hbm, bias_hbm, out_hbm,
             idxv, wv, rows0, rows1, outr, biasv, sem0, sem1, semo):
    cid = lax.axis_index("c")
    sid = lax.axis_index("s")
    wid = sid * NC + cid
    base = wid * BPW
    pltpu.sync_copy(ridx2_hbm.at[pl.ds(2 * base, 2 * BPW)], idxv)
    pltpu.sync_copy(wts_hbm.at[pl.ds(base, BPW)], wv)
    pltpu.sync_copy(bias_hbm, biasv)

    pltpu.async_copy(table_hbm.at[idxv.at[0]], rows0, sem0)

    @pl.loop(0, 2 * BPW, step=2)
    def _g_loop(g):
        b = g // 2
        pltpu.async_copy(table_hbm.at[idxv.at[g + 1]], rows1, sem1)
        pltpu.make_async_copy(table_hbm.at[idxv.at[g]], rows0, sem0).wait()

        @pl.when(g > 0)
        def _drain_prev():
            pltpu.make_async_copy(outr, out_hbm.at[base + b - 1], semo).wait()

        _accum_half(b, 0, rows0, wv, biasv, outr)

        @pl.when(g + 2 < 2 * BPW)
        def _issue_next():
            pltpu.async_copy(table_hbm.at[idxv.at[g + 2]], rows0, sem0)

        pltpu.make_async_copy(table_hbm.at[idxv.at[g + 1]], rows1, sem1).wait()
        _accum_half(b, 1, rows1, wv, biasv, outr)
        pltpu.async_copy(outr, out_hbm.at[base + b], semo)

    pltpu.make_async_copy(outr, out_hbm.at[base + BPW - 1], semo).wait()


def _tc_eval_body(xT_ref, y2_ref, d2_ref, kn_ref, b_ref, out_ref):
    xb = xT_ref[...]
    k0 = kn_ref[0]
    kN = kn_ref[K - 1]
    s = (kN - k0) / (K - 1)
    xc = jnp.clip(xb, k0, kN)
    u = (xc - k0) / s
    idxf = jnp.clip(jnp.floor(u), 0.0, K - 2)
    t = u - idxf
    t2 = t * t
    t3 = t2 * t
    h00 = 2 * t3 - 3 * t2 + 1
    h10 = t3 - 2 * t2 + t
    h01 = -2 * t3 + 3 * t2
    h11 = t3 - t2
    a0 = h00
    a1 = h01
    b0 = h10 * s
    b1 = h11 * s
    kio = jax.lax.broadcasted_iota(jnp.int32, (D_IN, K, TC_BB), 1)
    idxe = idxf.astype(jnp.int32)[:, None, :]
    e0 = kio == idxe
    e1 = kio == idxe + 1
    zero = jnp.zeros((), jnp.float32)
    W0 = jnp.where(e0, a0[:, None, :], zero) + jnp.where(e1, a1[:, None, :], zero)
    W1 = jnp.where(e0, b0[:, None, :], zero) + jnp.where(e1, b1[:, None, :], zero)
    W0 = W0.reshape(D_IN * K, TC_BB)
    W1 = W1.reshape(D_IN * K, TC_BB)
    acc = jnp.dot(y2_ref[...], W0, preferred_element_type=jnp.float32)
    acc = acc + jnp.dot(d2_ref[...], W1, preferred_element_type=jnp.float32)
    out_ref[...] = acc + b_ref[...]


def kernel(x, coeffs, bias, knots):
    slopes = pl.pallas_call(
        _slopes_body,
        out_shape=jax.ShapeDtypeStruct((D_OUT, D_IN, K), jnp.float32),
        in_specs=[
            pl.BlockSpec(memory_space=pltpu.VMEM),
            pl.BlockSpec(memory_space=pltpu.SMEM),
        ],
        out_specs=pl.BlockSpec(memory_space=pltpu.VMEM),
    )(coeffs, knots)

    ridx, wts = pl.pallas_call(
        _prep_body,
        grid=(BSC // PREP_BB,),
        in_specs=[
            pl.BlockSpec((PREP_BB, D_IN), lambda j: (j, 0)),
            pl.BlockSpec(memory_space=pltpu.SMEM),
        ],
        out_specs=[
            pl.BlockSpec((PREP_BB, D_IN), lambda j: (j, 0)),
            pl.BlockSpec((PREP_BB, 3, D_IN), lambda j: (j, 0, 0)),
        ],
        out_shape=[
            jax.ShapeDtypeStruct((BSC, D_IN), jnp.int32),
            jax.ShapeDtypeStruct((BSC, 3, D_IN), jnp.float32),
        ],
    )(x[:BSC], knots)

    yio = coeffs.transpose(1, 2, 0)
    dio = slopes.transpose(1, 2, 0)
    yn = jnp.roll(yio, -1, axis=1)
    dn = jnp.roll(dio, -1, axis=1)
    table = jnp.concatenate([yn, yio - yn, dio, dn], axis=2).reshape(
        D_IN * K, 4 * D_OUT)

    sc_call = pl.kernel(
        _sc_body,
        out_type=jax.ShapeDtypeStruct((BSC, D_OUT), jnp.float32),
        mesh=plsc.VectorSubcoreMesh(
            core_axis_name="c", subcore_axis_name="s",
            num_cores=NC, num_subcores=NS),
        scratch_types=[
            pltpu.VMEM((2 * BPW, HALF), jnp.int32),
            pltpu.VMEM((BPW, 3, D_IN), jnp.float32),
            pltpu.VMEM((HALF, 4 * D_OUT), jnp.float32),
            pltpu.VMEM((HALF, 4 * D_OUT), jnp.float32),
            pltpu.VMEM((D_OUT,), jnp.float32),
            pltpu.VMEM((D_OUT,), jnp.float32),
            pltpu.SemaphoreType.DMA,
            pltpu.SemaphoreType.DMA,
            pltpu.SemaphoreType.DMA,
        ],
    )
    sc_out = sc_call(ridx.reshape(2 * BSC, HALF), wts, table, bias)

    xT = x[BSC:].T
    y2 = coeffs.reshape(D_OUT, D_IN * K)
    d2 = slopes.reshape(D_OUT, D_IN * K)
    bias2 = bias.reshape(D_OUT, 1)
    tc_outT = pl.pallas_call(
        _tc_eval_body,
        grid=(BTC // TC_BB,),
        in_specs=[
            pl.BlockSpec((D_IN, TC_BB), lambda j: (0, j)),
            pl.BlockSpec((D_OUT, D_IN * K), lambda j: (0, 0)),
            pl.BlockSpec((D_OUT, D_IN * K), lambda j: (0, 0)),
            pl.BlockSpec(memory_space=pltpu.SMEM),
            pl.BlockSpec((D_OUT, 1), lambda j: (0, 0)),
        ],
        out_specs=pl.BlockSpec((D_OUT, TC_BB), lambda j: (0, j)),
        out_shape=jax.ShapeDtypeStruct((D_OUT, BTC), jnp.float32),
    )(xT, y2, d2, knots, bias2)
    return jnp.concatenate([sc_out, tc_outT.T], axis=0)


# speedup vs baseline: 7.2491x; 1.0072x over previous
import functools

import jax
import jax.numpy as jnp
from jax import lax
from jax.experimental import pallas as pl
from jax.experimental.pallas import tpu as pltpu
from jax.experimental.pallas import tpu_sc as plsc

D_IN = 64
D_OUT = 64
K = 64
B = 4096

NC = 2
NS = 16
NW = NC * NS

BSC = 256
BTC = B - BSC
BPW = BSC // NW

PREP_BB = min(512, BSC)
TC_BB = 512 if BTC % 512 == 0 else 256


def _slopes_body(y_ref, kn_ref, d_ref):
    y = y_ref[...]
    k0 = kn_ref[0]
    kN = kn_ref[K - 1]
    s = (kN - k0) / (K - 1)
    delta = (y[..., 1:] - y[..., :-1]) / (s + 1e-12)
    d0 = (3 * s * delta[..., 0] - s * delta[..., 1]) / (2 * s + 1e-12)
    dN = (3 * s * delta[..., -1] - s * delta[..., -2]) / (2 * s + 1e-12)

    def limit(di, deltai):
        di = jnp.where(di * deltai <= 0, jnp.zeros_like(di), di)
        di = jnp.where(jnp.abs(di) > 3 * jnp.abs(deltai), 3 * deltai, di)
        return di

    d0 = limit(d0, delta[..., 0])
    dN = limit(dN, delta[..., -1])
    dp = delta[..., :-1]
    dn = delta[..., 1:]
    same_sign = dp * dn > 0
    w = 3 * s
    d_int = (2 * w) / (w / (dp + 1e-12) + w / (dn + 1e-12) + 1e-12)
    d_int = jnp.where(same_sign, d_int, jnp.zeros_like(d_int))
    d_ref[...] = jnp.concatenate([d0[..., None], d_int, dN[..., None]], axis=-1)


def _prep_body(x_ref, kn_ref, ridx_ref, wts_ref):
    xb = x_ref[...]
    k0 = kn_ref[0]
    kN = kn_ref[K - 1]
    s = (kN - k0) / (K - 1)
    xc = jnp.clip(xb, k0, kN)
    u = (xc - k0) / s
    idxf = jnp.clip(jnp.floor(u), 0.0, K - 2)
    t = u - idxf
    t2 = t * t
    t3 = t2 * t
    h00 = 2 * t3 - 3 * t2 + 1
    h10 = t3 - 2 * t2 + t
    h11 = t3 - t2
    iio = lax.broadcasted_iota(jnp.int32, xb.shape, 1)
    ridx_ref[...] = iio * K + idxf.astype(jnp.int32)
    wts_ref[...] = jnp.concatenate(
        [h00[:, None, :], (h10 * s)[:, None, :], (h11 * s)[:, None, :]], axis=1)


HALF = D_IN // 2


def _accum_half(b, half, rows, wv, biasv, outr):
    zero = jnp.zeros((16,), jnp.float32)
    if half == 0:
        init = [[biasv[pl.ds(16 * c, 16)], zero] for c in range(4)]
    else:
        init = [[outr[pl.ds(16 * c, 16)], zero] for c in range(4)]

    @pl.loop(0, HALF // 16, init_carry=init)
    def _ci_loop(ci, accs):
        w0v = wv[b, 0, pl.ds(half * HALF + ci * 16, 16)]
        w1v = wv[b, 1, pl.ds(half * HALF + ci * 16, 16)]
        w2v = wv[b, 2, pl.ds(half * HALF + ci * 16, 16)]
        ibase = ci * 16
        for il in range(16):
            p = il % 2
            w0 = w0v[il]
            w1 = w1v[il]
            w2 = w2v[il]
            for c in range(4):
                r0 = rows[ibase + il, pl.ds(c * 16, 16)]
                r1 = rows[ibase + il, pl.ds(64 + c * 16, 16)]
                r2 = rows[ibase + il, pl.ds(128 + c * 16, 16)]
                r3 = rows[ibase + il, pl.ds(192 + c * 16, 16)]
                s1 = r0 + w0 * r1
                s2 = w1 * r2 + w2 * r3
                accs[c][p] = accs[c][p] + (s1 + s2)
        return accs

    accs = _ci_loop
    for c in range(4):
        outr[pl.ds(16 * c, 16)] = accs[c][0] + accs[c][1]


def _sc_body(ridx2_hbm, wts_hbm, table_hbm, bias_hbm, out_hbm,
             idxv, wv, rows0, rows1, outr, biasv, sem0, sem1, semo):
    cid = lax.axis_index("c")
    sid = lax.axis_index("s")
    wid = sid * NC + cid
    base = wid * BPW
    pltpu.sync_copy(ridx2_hbm.at[pl.ds(2 * base, 2 * BPW)], idxv)
    pltpu.sync_copy(wts_hbm.at[pl.ds(base, BPW)], wv)
    pltpu.sync_copy(bias_hbm, biasv)

    pltpu.async_copy(table_hbm.at[idxv.at[0]], rows0, sem0)

    @pl.loop(0, 2 * BPW, step=2)
    def _g_loop(g):
        b = g // 2
        pltpu.async_copy(table_hbm.at[idxv.at[g + 1]], rows1, sem1)
        pltpu.make_async_copy(table_hbm.at[idxv.at[g]], rows0, sem0).wait()

        @pl.when(g > 0)
        def _drain_prev():
            pltpu.make_async_copy(outr, out_hbm.at[base + b - 1], semo).wait()

        _accum_half(b, 0, rows0, wv, biasv, outr)

        @pl.when(g + 2 < 2 * BPW)
        def _issue_next():
            pltpu.async_copy(table_hbm.at[idxv.at[g + 2]], rows0, sem0)

        pltpu.make_async_copy(table_hbm.at[idxv.at[g + 1]], rows1, sem1).wait()
        _accum_half(b, 1, rows1, wv, biasv, outr)
        pltpu.async_copy(outr, out_hbm.at[base + b], semo)

    pltpu.make_async_copy(outr, out_hbm.at[base + BPW - 1], semo).wait()


def _tc_eval_body(xT_ref, y2_ref, d2_ref, kn_ref, b_ref, out_ref):
    xb = xT_ref[...]
    k0 = kn_ref[0]
    kN = kn_ref[K - 1]
    s = (kN - k0) / (K - 1)
    xc = jnp.clip(xb, k0, kN)
    u = (xc - k0) / s
    idxf = jnp.clip(jnp.floor(u), 0.0, K - 2)
    t = u - idxf
    t2 = t * t
    t3 = t2 * t
    h00 = 2 * t3 - 3 * t2 + 1
    h10 = t3 - 2 * t2 + t
    h01 = -2 * t3 + 3 * t2
    h11 = t3 - t2
    a0 = h00
    a1 = h01
    b0 = h10 * s
    b1 = h11 * s
    kio = jax.lax.broadcasted_iota(jnp.int32, (D_IN, K, TC_BB), 1)
    idxe = idxf.astype(jnp.int32)[:, None, :]
    e0 = kio == idxe
    e1 = kio == idxe + 1
    zero = jnp.zeros((), jnp.float32)
    W0 = jnp.where(e0, a0[:, None, :], zero) + jnp.where(e1, a1[:, None, :], zero)
    W1 = jnp.where(e0, b0[:, None, :], zero) + jnp.where(e1, b1[:, None, :], zero)
    W0 = W0.reshape(D_IN * K, TC_BB)
    W1 = W1.reshape(D_IN * K, TC_BB)
    acc = jnp.dot(y2_ref[...], W0, preferred_element_type=jnp.float32)
    acc = acc + jnp.dot(d2_ref[...], W1, preferred_element_type=jnp.float32)
    out_ref[...] = acc + b_ref[...]


def kernel(x, coeffs, bias, knots):
    slopes = pl.pallas_call(
        _slopes_body,
        out_shape=jax.ShapeDtypeStruct((D_OUT, D_IN, K), jnp.float32),
        in_specs=[
            pl.BlockSpec(memory_space=pltpu.VMEM),
            pl.BlockSpec(memory_space=pltpu.SMEM),
        ],
        out_specs=pl.BlockSpec(memory_space=pltpu.VMEM),
    )(coeffs, knots)

    ridx, wts = pl.pallas_call(
        _prep_body,
        grid=(BSC // PREP_BB,),
        in_specs=[
            pl.BlockSpec((PREP_BB, D_IN), lambda j: (j, 0)),
            pl.BlockSpec(memory_space=pltpu.SMEM),
        ],
        out_specs=[
            pl.BlockSpec((PREP_BB, D_IN), lambda j: (j, 0)),
            pl.BlockSpec((PREP_BB, 3, D_IN), lambda j: (j, 0, 0)),
        ],
        out_shape=[
            jax.ShapeDtypeStruct((BSC, D_IN), jnp.int32),
            jax.ShapeDtypeStruct((BSC, 3, D_IN), jnp.float32),
        ],
    )(x[:BSC], knots)

    yio = coeffs.transpose(1, 2, 0)
    dio = slopes.transpose(1, 2, 0)
    yn = jnp.roll(yio, -1, axis=1)
    dn = jnp.roll(dio, -1, axis=1)
    table = jnp.concatenate([yn, yio - yn, dio, dn], axis=2).reshape(
        D_IN * K, 4 * D_OUT)

    sc_call = pl.kernel(
        _sc_body,
        out_type=jax.ShapeDtypeStruct((BSC, D_OUT), jnp.float32),
        mesh=plsc.VectorSubcoreMesh(
            core_axis_name="c", subcore_axis_name="s",
            num_cores=NC, num_subcores=NS),
        scratch_types=[
            pltpu.VMEM((2 * BPW, HALF), jnp.int32),
            pltpu.VMEM((BPW, 3, D_IN), jnp.float32),
            pltpu.VMEM((HALF, 4 * D_OUT), jnp.float32),
            pltpu.VMEM((HALF, 4 * D_OUT), jnp.float32),
            pltpu.VMEM((D_OUT,), jnp.float32),
            pltpu.VMEM((D_OUT,), jnp.float32),
            pltpu.SemaphoreType.DMA,
            pltpu.SemaphoreType.DMA,
            pltpu.SemaphoreType.DMA,
        ],
    )
    sc_out = sc_call(ridx.reshape(2 * BSC, HALF), wts, table, bias)

    xT = x[BSC:].T
    y2 = coeffs.reshape(D_OUT, D_IN * K)
    d2 = slopes.reshape(D_OUT, D_IN * K)
    bias2 = bias.reshape(D_OUT, 1)
    tc_outT = pl.pallas_call(
        _tc_eval_body,
        grid=(BTC // TC_BB,),
        in_specs=[
            pl.BlockSpec((D_IN, TC_BB), lambda j: (0, j)),
            pl.BlockSpec((D_OUT, D_IN * K), lambda j: (0, 0)),
            pl.BlockSpec((D_OUT, D_IN * K), lambda j: (0, 0)),
            pl.BlockSpec(memory_space=pltpu.SMEM),
            pl.BlockSpec((D_OUT, 1), lambda j: (0, 0)),
        ],
        out_specs=pl.BlockSpec((D_OUT, TC_BB), lambda j: (0, j)),
        out_shape=jax.ShapeDtypeStruct((D_OUT, BTC), jnp.float32),
    )(xT, y2, d2, knots, bias2)
    return jnp.concatenate([sc_out, tc_outT.T], axis=0)


# speedup vs baseline: 7.4692x; 1.0304x over previous
import functools

import jax
import jax.numpy as jnp
from jax import lax
from jax.experimental import pallas as pl
from jax.experimental.pallas import tpu as pltpu
from jax.experimental.pallas import tpu_sc as plsc

D_IN = 64
D_OUT = 64
K = 64
B = 4096

NC = 2
NS = 16
NW = NC * NS

BSC = 512
BTC = B - BSC
BPW = BSC // NW

PREP_BB = min(512, BSC)
TC_BB = 512 if BTC % 512 == 0 else 256


def _slopes_body(y_ref, kn_ref, d_ref):
    y = y_ref[...]
    k0 = kn_ref[0]
    kN = kn_ref[K - 1]
    s = (kN - k0) / (K - 1)
    delta = (y[..., 1:] - y[..., :-1]) / (s + 1e-12)
    d0 = (3 * s * delta[..., 0] - s * delta[..., 1]) / (2 * s + 1e-12)
    dN = (3 * s * delta[..., -1] - s * delta[..., -2]) / (2 * s + 1e-12)

    def limit(di, deltai):
        di = jnp.where(di * deltai <= 0, jnp.zeros_like(di), di)
        di = jnp.where(jnp.abs(di) > 3 * jnp.abs(deltai), 3 * deltai, di)
        return di

    d0 = limit(d0, delta[..., 0])
    dN = limit(dN, delta[..., -1])
    dp = delta[..., :-1]
    dn = delta[..., 1:]
    same_sign = dp * dn > 0
    w = 3 * s
    d_int = (2 * w) / (w / (dp + 1e-12) + w / (dn + 1e-12) + 1e-12)
    d_int = jnp.where(same_sign, d_int, jnp.zeros_like(d_int))
    d_ref[...] = jnp.concatenate([d0[..., None], d_int, dN[..., None]], axis=-1)


def _prep_body(x_ref, kn_ref, ridx_ref, wts_ref):
    xb = x_ref[...]
    k0 = kn_ref[0]
    kN = kn_ref[K - 1]
    s = (kN - k0) / (K - 1)
    xc = jnp.clip(xb, k0, kN)
    u = (xc - k0) / s
    idxf = jnp.clip(jnp.floor(u), 0.0, K - 2)
    t = u - idxf
    t2 = t * t
    t3 = t2 * t
    h00 = 2 * t3 - 3 * t2 + 1
    h10 = t3 - 2 * t2 + t
    h11 = t3 - t2
    iio = lax.broadcasted_iota(jnp.int32, xb.shape, 1)
    ridx_ref[...] = iio * K + idxf.astype(jnp.int32)
    wts_ref[...] = jnp.concatenate(
        [h00[:, None, :], (h10 * s)[:, None, :], (h11 * s)[:, None, :]], axis=1)


HALF = D_IN // 2


def _accum_half(b, half, rows, wv, biasv, outr):
    zero = jnp.zeros((16,), jnp.float32)
    if half == 0:
        init = [[biasv[pl.ds(16 * c, 16)], zero] for c in range(4)]
    else:
        init = [[outr[pl.ds(16 * c, 16)], zero] for c in range(4)]

    @pl.loop(0, HALF // 16, init_carry=init)
    def _ci_loop(ci, accs):
        w0v = wv[b, 0, pl.ds(half * HALF + ci * 16, 16)]
        w1v = wv[b, 1, pl.ds(half * HALF + ci * 16, 16)]
        w2v = wv[b, 2, pl.ds(half * HALF + ci * 16, 16)]
        ibase = ci * 16
        for il in range(16):
            p = il % 2
            w0 = w0v[il]
            w1 = w1v[il]
            w2 = w2v[il]
            for c in range(4):
                r0 = rows[ibase + il, pl.ds(c * 16, 16)]
                r1 = rows[ibase + il, pl.ds(64 + c * 16, 16)]
                r2 = rows[ibase + il, pl.ds(128 + c * 16, 16)]
                r3 = rows[ibase + il, pl.ds(192 + c * 16, 16)]
                s1 = r0 + w0 * r1
                s2 = w1 * r2 + w2 * r3
                accs[c][p] = accs[c][p] + (s1 + s2)
        return accs

    accs = _ci_loop
    for c in range(4):
        outr[pl.ds(16 * c, 16)] = accs[c][0] + accs[c][1]


def _sc_body(ridx2_hbm, wts_hbm, table_hbm, bias_hbm, out_hbm,
             idxv, wv, rows0, rows1, outr, biasv, sem0, sem1, semo):
    cid = lax.axis_index("c")
    sid = lax.axis_index("s")
    wid = sid * NC + cid
    base = wid * BPW
    pltpu.sync_copy(ridx2_hbm.at[pl.ds(2 * base, 2 * BPW)], idxv)
    pltpu.sync_copy(wts_hbm.at[pl.ds(base, BPW)], wv)
    pltpu.sync_copy(bias_hbm, biasv)

    pltpu.async_copy(table_hbm.at[idxv.at[0]], rows0, sem0)

    @pl.loop(0, 2 * BPW, step=2)
    def _g_loop(g):
        b = g // 2
        pltpu.async_copy(table_hbm.at[idxv.at[g + 1]], rows1, sem1)
        pltpu.make_async_copy(table_hbm.at[idxv.at[g]], rows0, sem0).wait()

        @pl.when(g > 0)
        def _drain_prev():
            pltpu.make_async_copy(outr, out_hbm.at[base + b - 1], semo).wait()

        _accum_half(b, 0, rows0, wv, biasv, outr)

        @pl.when(g + 2 < 2 * BPW)
        def _issue_next():
            pltpu.async_copy(table_hbm.at[idxv.at[g + 2]], rows0, sem0)

        pltpu.make_async_copy(table_hbm.at[idxv.at[g + 1]], rows1, sem1).wait()
        _accum_half(b, 1, rows1, wv, biasv, outr)
        pltpu.async_copy(outr, out_hbm.at[base + b], semo)

    pltpu.make_async_copy(outr, out_hbm.at[base + BPW - 1], semo).wait()


def _tc_eval_body(xT_ref, y2_ref, d2_ref, kn_ref, b_ref, out_ref):
    xb = xT_ref[...]
    k0 = kn_ref[0]
    kN = kn_ref[K - 1]
    s = (kN - k0) / (K - 1)
    xc = jnp.clip(xb, k0, kN)
    u = (xc - k0) / s
    idxf = jnp.clip(jnp.floor(u), 0.0, K - 2)
    t = u - idxf
    t2 = t * t
    t3 = t2 * t
    h00 = 2 * t3 - 3 * t2 + 1
    h10 = t3 - 2 * t2 + t
    h01 = -2 * t3 + 3 * t2
    h11 = t3 - t2
    a0 = h00
    a1 = h01
    b0 = h10 * s
    b1 = h11 * s
    kio = jax.lax.broadcasted_iota(jnp.int32, (D_IN, K, TC_BB), 1)
    idxe = idxf.astype(jnp.int32)[:, None, :]
    e0 = kio == idxe
    e1 = kio == idxe + 1
    zero = jnp.zeros((), jnp.float32)
    W0 = jnp.where(e0, a0[:, None, :], jnp.where(e1, a1[:, None, :], zero))
    W1 = jnp.where(e0, b0[:, None, :], jnp.where(e1, b1[:, None, :], zero))
    W0 = W0.reshape(D_IN * K, TC_BB)
    W1 = W1.reshape(D_IN * K, TC_BB)
    acc = jnp.dot(y2_ref[...], W0, preferred_element_type=jnp.float32)
    acc = acc + jnp.dot(d2_ref[...], W1, preferred_element_type=jnp.float32)
    out_ref[...] = acc + b_ref[...]


def kernel(x, coeffs, bias, knots):
    slopes = pl.pallas_call(
        _slopes_body,
        out_shape=jax.ShapeDtypeStruct((D_OUT, D_IN, K), jnp.float32),
        in_specs=[
            pl.BlockSpec(memory_space=pltpu.VMEM),
            pl.BlockSpec(memory_space=pltpu.SMEM),
        ],
        out_specs=pl.BlockSpec(memory_space=pltpu.VMEM),
    )(coeffs, knots)

    ridx, wts = pl.pallas_call(
        _prep_body,
        grid=(BSC // PREP_BB,),
        in_specs=[
            pl.BlockSpec((PREP_BB, D_IN), lambda j: (j, 0)),
            pl.BlockSpec(memory_space=pltpu.SMEM),
        ],
        out_specs=[
            pl.BlockSpec((PREP_BB, D_IN), lambda j: (j, 0)),
            pl.BlockSpec((PREP_BB, 3, D_IN), lambda j: (j, 0, 0)),
        ],
        out_shape=[
            jax.ShapeDtypeStruct((BSC, D_IN), jnp.int32),
            jax.ShapeDtypeStruct((BSC, 3, D_IN), jnp.float32),
        ],
    )(x[:BSC], knots)

    yio = coeffs.transpose(1, 2, 0)
    dio = slopes.transpose(1, 2, 0)
    yn = jnp.roll(yio, -1, axis=1)
    dn = jnp.roll(dio, -1, axis=1)
    table = jnp.concatenate([yn, yio - yn, dio, dn], axis=2).reshape(
        D_IN * K, 4 * D_OUT)

    sc_call = pl.kernel(
        _sc_body,
        out_type=jax.ShapeDtypeStruct((BSC, D_OUT), jnp.float32),
        mesh=plsc.VectorSubcoreMesh(
            core_axis_name="c", subcore_axis_name="s",
            num_cores=NC, num_subcores=NS),
        scratch_types=[
            pltpu.VMEM((2 * BPW, HALF), jnp.int32),
            pltpu.VMEM((BPW, 3, D_IN), jnp.float32),
            pltpu.VMEM((HALF, 4 * D_OUT), jnp.float32),
            pltpu.VMEM((HALF, 4 * D_OUT), jnp.float32),
            pltpu.VMEM((D_OUT,), jnp.float32),
            pltpu.VMEM((D_OUT,), jnp.float32),
            pltpu.SemaphoreType.DMA,
            pltpu.SemaphoreType.DMA,
            pltpu.SemaphoreType.DMA,
        ],
    )
    sc_out = sc_call(ridx.reshape(2 * BSC, HALF), wts, table, bias)

    xT = x[BSC:].T
    y2 = coeffs.reshape(D_OUT, D_IN * K)
    d2 = slopes.reshape(D_OUT, D_IN * K)
    bias2 = bias.reshape(D_OUT, 1)
    tc_outT = pl.pallas_call(
        _tc_eval_body,
        grid=(BTC // TC_BB,),
        in_specs=[
            pl.BlockSpec((D_IN, TC_BB), lambda j: (0, j)),
            pl.BlockSpec((D_OUT, D_IN * K), lambda j: (0, 0)),
            pl.BlockSpec((D_OUT, D_IN * K), lambda j: (0, 0)),
            pl.BlockSpec(memory_space=pltpu.SMEM),
            pl.BlockSpec((D_OUT, 1), lambda j: (0, 0)),
        ],
        out_specs=pl.BlockSpec((D_OUT, TC_BB), lambda j: (0, j)),
        out_shape=jax.ShapeDtypeStruct((D_OUT, BTC), jnp.float32),
    )(xT, y2, d2, knots, bias2)
    return jnp.concatenate([sc_out, tc_outT.T], axis=0)
